# baseline (speedup 1.0000x reference)
import jax
import jax.numpy as jnp
from jax.experimental import pallas as pl
from jax.experimental.pallas import tpu as pltpu

N = 4096
D = 1024
C = 128
NB = 16
BLK = N // NB


def _ge2e_kernel(emb_hbm, y_ref, w_ref, b_ref, out_ref, e_scr, ebf_scr, sem):
    cps = [
        pltpu.make_async_copy(
            emb_hbm.at[pl.ds(i * BLK, BLK), :],
            e_scr.at[pl.ds(i * BLK, BLK), :],
            sem.at[i],
        )
        for i in range(NB)
    ]
    for cp in cps:
        cp.start()

    yv = y_ref[...]
    kio = jax.lax.broadcasted_iota(jnp.int32, (C, N), 0)
    ohb = kio == yv
    counts = jnp.sum(jnp.where(ohb, 1.0, 0.0), axis=1, keepdims=True)
    n_y = jnp.sum(jnp.where(ohb, counts, 0.0), axis=0, keepdims=True)
    inv_n = 1.0 / counts
    ones_bf = jnp.ones((1, D), dtype=jnp.bfloat16)

    cent = jnp.zeros((C, D), dtype=jnp.float32)
    rn2_parts = []
    for i in range(NB):
        cps[i].wait()
        Eb = e_scr[pl.ds(i * BLK, BLK), :]
        Esq_bf = (Eb * Eb).astype(jnp.bfloat16)
        rn2_b = jax.lax.dot_general(
            ones_bf, Esq_bf, (((1,), (1,)), ((), ())),
            preferred_element_type=jnp.float32)
        rn2_parts.append(rn2_b)
        Ebf_b = Eb.astype(jnp.bfloat16)
        ebf_scr[pl.ds(i * BLK, BLK), :] = Ebf_b
        inv_rb = 1.0 / jnp.maximum(jnp.sqrt(rn2_b), 1e-12)
        ohb_b = ohb[:, i * BLK:(i + 1) * BLK]
        ohs_bf = jnp.where(ohb_b, inv_rb, 0.0).astype(jnp.bfloat16)
        cent = cent + jax.lax.dot_general(
            ohs_bf, Ebf_b, (((1,), (0,)), ((), ())),
            preferred_element_type=jnp.float32)

    rn2_row = jnp.concatenate(rn2_parts, axis=1)
    rn_row = jnp.sqrt(rn2_row)
    inv_r = 1.0 / jnp.maximum(rn_row, 1e-12)
    se_row = rn_row * inv_r
    se2_row = se_row * se_row
    inv_ne = 1.0 / jnp.maximum(se_row, 1e-8)

    csq = jnp.sum(cent * cent, axis=1, keepdims=True)
    norm_co = jnp.maximum(jnp.sqrt(csq) * inv_n, 1e-8)
    A = inv_n / norm_co

    GTr = jax.lax.dot_general(
        cent.astype(jnp.bfloat16), ebf_scr[...], (((1,), (1,)), ((), ())),
        preferred_element_type=jnp.float32)

    w = w_ref[0]
    ccw = (inv_r * inv_ne) * w

    Gdiag = jnp.sum(jnp.where(ohb, GTr, 0.0), axis=0, keepdims=True) * inv_r
    csq_y = jnp.sum(jnp.where(ohb, csq, 0.0), axis=0, keepdims=True)

    inv_nm1 = 1.0 / (n_y - 1.0)
    num_own = (Gdiag - se2_row) * inv_nm1
    own_sq = jnp.maximum(csq_y - 2.0 * Gdiag + se2_row, 0.0)
    norm_own = jnp.maximum(jnp.sqrt(own_sq) * inv_nm1, 1e-8)
    S_own = num_own * inv_ne / norm_own
    XOwn = S_own * w

    X = jnp.where(ohb, XOwn, (GTr * ccw) * A)

    m = jnp.max(X, axis=1, keepdims=True)
    lse = jnp.log(jnp.sum(jnp.exp(X - m), axis=1, keepdims=True)) + m

    t1 = jnp.sum(counts * lse, axis=0, keepdims=True)
    t2 = jnp.sum(XOwn, axis=1, keepdims=True)
    out_ref[...] = t1 - t2


@jax.jit
def _ge2e(emb, w, b, y):
    y2 = y.astype(jnp.int32).reshape(1, N)
    out = pl.pallas_call(
        _ge2e_kernel,
        out_shape=jax.ShapeDtypeStruct((1, 1), jnp.float32),
        in_specs=[
            pl.BlockSpec(memory_space=pltpu.MemorySpace.HBM),
            pl.BlockSpec(memory_space=pltpu.VMEM),
            pl.BlockSpec(memory_space=pltpu.SMEM),
            pl.BlockSpec(memory_space=pltpu.SMEM),
        ],
        out_specs=pl.BlockSpec(memory_space=pltpu.VMEM),
        scratch_shapes=[
            pltpu.VMEM((N, D), jnp.float32),
            pltpu.VMEM((N, D), jnp.bfloat16),
            pltpu.SemaphoreType.DMA((NB,)),
        ],
    )(emb, y2, w.reshape(1), b.reshape(1))
    return out[0, 0]


def kernel(emb, w, b, y):
    return _ge2e(emb, w, b, y)


# speedup vs baseline: 1.2129x; 1.2129x over previous
import jax
import jax.numpy as jnp
from jax.experimental import pallas as pl
from jax.experimental.pallas import tpu as pltpu

N = 4096
D = 1024
C = 128
NB = 8
BLK = N // NB


def _ge2e_kernel(emb_hbm, y_ref, w_ref, b_ref, out_ref, e_scr, ebf_scr, sem):
    cps = [
        pltpu.make_async_copy(
            emb_hbm.at[pl.ds(i * BLK, BLK), :],
            e_scr.at[pl.ds(i * BLK, BLK), :],
            sem.at[i],
        )
        for i in range(NB)
    ]
    for cp in cps:
        cp.start()

    yv = y_ref[...]
    kio = jax.lax.broadcasted_iota(jnp.int32, (C, N), 0)
    ohb = kio == yv
    counts = jnp.sum(jnp.where(ohb, 1.0, 0.0), axis=1, keepdims=True)
    n_y = jnp.sum(jnp.where(ohb, counts, 0.0), axis=0, keepdims=True)
    inv_n = 1.0 / counts
    ones_bf = jnp.ones((1, D), dtype=jnp.bfloat16)

    cent = jnp.zeros((C, D), dtype=jnp.float32)
    rn2_parts = []
    for i in range(NB):
        cps[i].wait()
        Ebf_b = e_scr[pl.ds(i * BLK, BLK), :].astype(jnp.bfloat16)
        ebf_scr[pl.ds(i * BLK, BLK), :] = Ebf_b
        rn2_b = jax.lax.dot_general(
            ones_bf, Ebf_b * Ebf_b, (((1,), (1,)), ((), ())),
            preferred_element_type=jnp.float32)
        rn2_parts.append(rn2_b)
        inv_rb = 1.0 / jnp.maximum(jnp.sqrt(rn2_b), 1e-12)
        ohb_b = ohb[:, i * BLK:(i + 1) * BLK]
        ohs_bf = jnp.where(ohb_b, inv_rb, 0.0).astype(jnp.bfloat16)
        cent = cent + jax.lax.dot_general(
            ohs_bf, Ebf_b, (((1,), (0,)), ((), ())),
            preferred_element_type=jnp.float32)

    rn2_row = jnp.concatenate(rn2_parts, axis=1)
    rn_row = jnp.sqrt(rn2_row)
    inv_r = 1.0 / jnp.maximum(rn_row, 1e-12)
    se_row = rn_row * inv_r
    se2_row = se_row * se_row
    inv_ne = 1.0 / jnp.maximum(se_row, 1e-8)

    csq = jnp.sum(cent * cent, axis=1, keepdims=True)
    norm_co = jnp.maximum(jnp.sqrt(csq) * inv_n, 1e-8)
    A = inv_n / norm_co

    GTr = jax.lax.dot_general(
        cent.astype(jnp.bfloat16), ebf_scr[...], (((1,), (1,)), ((), ())),
        preferred_element_type=jnp.float32)

    w = w_ref[0]
    ccw = (inv_r * inv_ne) * w

    Gdiag = jnp.sum(jnp.where(ohb, GTr, 0.0), axis=0, keepdims=True) * inv_r
    csq_y = jnp.sum(jnp.where(ohb, csq, 0.0), axis=0, keepdims=True)

    inv_nm1 = 1.0 / (n_y - 1.0)
    num_own = (Gdiag - se2_row) * inv_nm1
    own_sq = jnp.maximum(csq_y - 2.0 * Gdiag + se2_row, 0.0)
    norm_own = jnp.maximum(jnp.sqrt(own_sq) * inv_nm1, 1e-8)
    S_own = num_own * inv_ne / norm_own
    XOwn = S_own * w

    X = jnp.where(ohb, XOwn, (GTr * ccw) * A)

    m = jnp.max(X, axis=1, keepdims=True)
    lse = jnp.log(jnp.sum(jnp.exp(X - m), axis=1, keepdims=True)) + m

    t1 = jnp.sum(counts * lse, axis=0, keepdims=True)
    t2 = jnp.sum(XOwn, axis=1, keepdims=True)
    out_ref[...] = t1 - t2


@jax.jit
def _ge2e(emb, w, b, y):
    y2 = y.astype(jnp.int32).reshape(1, N)
    out = pl.pallas_call(
        _ge2e_kernel,
        out_shape=jax.ShapeDtypeStruct((1, 1), jnp.float32),
        in_specs=[
            pl.BlockSpec(memory_space=pltpu.MemorySpace.HBM),
            pl.BlockSpec(memory_space=pltpu.VMEM),
            pl.BlockSpec(memory_space=pltpu.SMEM),
            pl.BlockSpec(memory_space=pltpu.SMEM),
        ],
        out_specs=pl.BlockSpec(memory_space=pltpu.VMEM),
        scratch_shapes=[
            pltpu.VMEM((N, D), jnp.float32),
            pltpu.VMEM((N, D), jnp.bfloat16),
            pltpu.SemaphoreType.DMA((NB,)),
        ],
    )(emb, y2, w.reshape(1), b.reshape(1))
    return out[0, 0]


def kernel(emb, w, b, y):
    return _ge2e(emb, w, b, y)


# speedup vs baseline: 1.2182x; 1.0044x over previous
import jax
import jax.numpy as jnp
from jax.experimental import pallas as pl
from jax.experimental.pallas import tpu as pltpu

N = 4096
D = 1024
C = 128
NB = 8
BLK = N // NB


def _ge2e_kernel(emb_hbm, y_ref, w_ref, b_ref, out_ref, e_scr, ebf_scr, sem):
    cps = [
        pltpu.make_async_copy(
            emb_hbm.at[pl.ds(i * BLK, BLK), :],
            e_scr.at[pl.ds(i * BLK, BLK), :],
            sem.at[i],
        )
        for i in range(NB)
    ]
    for cp in cps:
        cp.start()

    yv = y_ref[...]
    kio = jax.lax.broadcasted_iota(jnp.int32, (C, N), 0)
    ohb = kio == yv
    counts = jnp.sum(jnp.where(ohb, 1.0, 0.0), axis=1, keepdims=True)
    n_y = jnp.sum(jnp.where(ohb, counts, 0.0), axis=0, keepdims=True)
    inv_n = 1.0 / counts
    ones_bf = jnp.ones((1, D), dtype=jnp.bfloat16)

    cent = jnp.zeros((C, D), dtype=jnp.float32)
    rn2_parts = []
    for i in range(NB):
        cps[i].wait()
        Ebf_b = e_scr[pl.ds(i * BLK, BLK), :].astype(jnp.bfloat16)
        ebf_scr[pl.ds(i * BLK, BLK), :] = Ebf_b
        rn2_b = jax.lax.dot_general(
            ones_bf, Ebf_b * Ebf_b, (((1,), (1,)), ((), ())),
            preferred_element_type=jnp.float32)
        rn2_parts.append(rn2_b)
        inv_rb = 1.0 / jnp.maximum(jnp.sqrt(rn2_b), 1e-12)
        ohb_b = ohb[:, i * BLK:(i + 1) * BLK]
        ohs_bf = jnp.where(ohb_b, inv_rb, 0.0).astype(jnp.bfloat16)
        cent = cent + jax.lax.dot_general(
            ohs_bf, Ebf_b, (((1,), (0,)), ((), ())),
            preferred_element_type=jnp.float32)

    rn2_row = jnp.concatenate(rn2_parts, axis=1)
    rn_row = jnp.sqrt(rn2_row)
    inv_r = 1.0 / jnp.maximum(rn_row, 1e-12)
    se_row = rn_row * inv_r
    se2_row = se_row * se_row
    inv_ne = 1.0 / jnp.maximum(se_row, 1e-8)

    csq = jnp.sum(cent * cent, axis=1, keepdims=True)
    norm_co = jnp.maximum(jnp.sqrt(csq) * inv_n, 1e-8)
    A = inv_n / norm_co

    GTr = jax.lax.dot_general(
        cent.astype(jnp.bfloat16), ebf_scr[...], (((1,), (1,)), ((), ())),
        preferred_element_type=jnp.float32)

    w = w_ref[0]
    ccw = (inv_r * inv_ne) * w

    Gdiag = jnp.sum(jnp.where(ohb, GTr, 0.0), axis=0, keepdims=True) * inv_r
    csq_y = jnp.sum(jnp.where(ohb, csq, 0.0), axis=0, keepdims=True)

    inv_nm1 = 1.0 / (n_y - 1.0)
    num_own = (Gdiag - se2_row) * inv_nm1
    own_sq = jnp.maximum(csq_y - 2.0 * Gdiag + se2_row, 0.0)
    norm_own = jnp.maximum(jnp.sqrt(own_sq) * inv_nm1, 1e-8)
    S_own = num_own * inv_ne / norm_own
    XOwn = S_own * w

    X = jnp.where(ohb, XOwn, (GTr * ccw) * A)

    m = jnp.max(X, axis=1, keepdims=True)
    lse = jnp.log(jnp.sum(jnp.exp(X - m), axis=1, keepdims=True)) + m

    t1 = jnp.sum(jnp.where(counts > 0.0, counts * lse, 0.0),
                 axis=0, keepdims=True)
    t2 = jnp.sum(XOwn, axis=1, keepdims=True)
    out_ref[...] = t1 - t2


@jax.jit
def _ge2e(emb, w, b, y):
    y2 = y.astype(jnp.int32).reshape(1, N)
    out = pl.pallas_call(
        _ge2e_kernel,
        out_shape=jax.ShapeDtypeStruct((1, 1), jnp.float32),
        in_specs=[
            pl.BlockSpec(memory_space=pltpu.MemorySpace.HBM),
            pl.BlockSpec(memory_space=pltpu.VMEM),
            pl.BlockSpec(memory_space=pltpu.SMEM),
            pl.BlockSpec(memory_space=pltpu.SMEM),
        ],
        out_specs=pl.BlockSpec(memory_space=pltpu.VMEM),
        scratch_shapes=[
            pltpu.VMEM((N, D), jnp.float32),
            pltpu.VMEM((N, D), jnp.bfloat16),
            pltpu.SemaphoreType.DMA((NB,)),
        ],
    )(emb, y2, w.reshape(1), b.reshape(1))
    return out[0, 0]


def kernel(emb, w, b, y):
    return _ge2e(emb, w, b, y)
